# baseline (speedup 1.0000x reference)
import functools

import jax
import jax.numpy as jnp
from jax import lax
from jax.experimental import pallas as pl
from jax.experimental.pallas import tpu as pltpu
from jax.experimental.pallas import tpu_sc as plsc

V = 100000
D = 32
B = 16384
NT = 10
NC = 2
NS = 16
NW = NC * NS
BPW = B // NW
CL = 128
CH = BPW // CL
NBUF = 8
L = 16

_mesh = plsc.VectorSubcoreMesh(core_axis_name="c", subcore_axis_name="s")


@functools.partial(
    pl.kernel,
    out_type=jax.ShapeDtypeStruct((B, NT * D), jnp.float32),
    mesh=_mesh,
    compiler_params=pltpu.CompilerParams(use_tc_tiling_on_sc=False,
                                         needs_layout_passes=False),
    scratch_types=(
        [pltpu.VMEM((BPW, NT), jnp.int32),
         pltpu.VMEM((NT, CH, CL), jnp.int32),
         pltpu.VMEM((NBUF, CL, D), jnp.float32)]
        + [pltpu.SemaphoreType.DMA] * (2 * NBUF)
    ),
)
def _emb_lookup(x_hbm, tcat, out_hbm, xv, idx_v, rows_v, *sems):
    wid = lax.axis_index("s") * NC + lax.axis_index("c")
    base = wid * BPW

    gsems = sems[:NBUF]
    ssems = sems[NBUF:]

    pltpu.sync_copy(x_hbm.at[pl.ds(base, BPW), :], xv)

    iota = lax.iota(jnp.int32, L)
    for i in range(NT):
        cols = jnp.full((L,), i, jnp.int32)
        for j in range(BPW // L):
            g = plsc.load_gather(xv, [iota + (j * L), cols]) + (i * V)
            c, jj = divmod(j, CL // L)
            idx_v[i, c, pl.ds(jj * L, L)] = g

    chunks = [(i, c) for i in range(NT) for c in range(CH)]
    n = len(chunks)
    gh = [None] * n
    sh = [None] * n

    def start_gather(k):
        i, c = chunks[k]
        b = k % NBUF
        gh[k] = pltpu.async_copy(
            tcat.at[idx_v.at[i, c]], rows_v.at[b], gsems[b])

    for k in range(min(NBUF, n)):
        start_gather(k)
    for k in range(n):
        i, c = chunks[k]
        b = k % NBUF
        gh[k].wait()
        sh[k] = pltpu.async_copy(
            rows_v.at[b],
            out_hbm.at[pl.ds(base + c * CL, CL), pl.ds(i * D, D)],
            ssems[b])
        if k + NBUF < n:
            sh[k].wait()
            start_gather(k + NBUF)
    for k in range(max(0, n - NBUF), n):
        sh[k].wait()


def kernel(x, W_count, W_fans, W_stars, W_hot, W_more, W_profile, W_cute,
           W_list, W_writer, W_photos):
    tcat = jnp.concatenate([W_count, W_fans, W_stars, W_hot, W_more,
                            W_profile, W_cute, W_list, W_writer, W_photos],
                           axis=0)
    return _emb_lookup(x.astype(jnp.int32), tcat)


# speedup vs baseline: 1.9568x; 1.9568x over previous
import functools

import jax
import jax.numpy as jnp
from jax import lax
from jax.experimental import pallas as pl
from jax.experimental.pallas import tpu as pltpu
from jax.experimental.pallas import tpu_sc as plsc

V = 100000
D = 32
B = 16384
NT = 10
NC = 2
NS = 16
NW = NC * NS
BPW = B // NW
CL = 128
CH = BPW // CL
NBUF = 8

_mesh = plsc.VectorSubcoreMesh(core_axis_name="c", subcore_axis_name="s")


@functools.partial(
    pl.kernel,
    out_type=jax.ShapeDtypeStruct((B, NT * D), jnp.float32),
    mesh=_mesh,
    compiler_params=pltpu.CompilerParams(use_tc_tiling_on_sc=False),
    scratch_types=(
        [pltpu.VMEM((NT, CH, CL), jnp.int32),
         pltpu.VMEM((NBUF, CL, D), jnp.float32)]
        + [pltpu.SemaphoreType.DMA] * (2 * NBUF)
    ),
)
def _emb_lookup(xr_hbm, t0, t1, t2, t3, t4, t5, t6, t7, t8, t9, out_hbm,
                idx_v, rows_v, *sems):
    wid = lax.axis_index("s") * NC + lax.axis_index("c")
    base = wid * BPW
    pltpu.sync_copy(xr_hbm.at[wid], idx_v)

    tables = [t0, t1, t2, t3, t4, t5, t6, t7, t8, t9]
    gsems = sems[:NBUF]
    ssems = sems[NBUF:]
    chunks = [(i, c) for i in range(NT) for c in range(CH)]
    n = len(chunks)
    gh = [None] * n
    sh = [None] * n

    def start_gather(k):
        i, c = chunks[k]
        b = k % NBUF
        gh[k] = pltpu.async_copy(
            tables[i].at[idx_v.at[i, c]], rows_v.at[b], gsems[b])

    for k in range(min(NBUF, n)):
        start_gather(k)
    for k in range(n):
        i, c = chunks[k]
        b = k % NBUF
        gh[k].wait()
        sh[k] = pltpu.async_copy(
            rows_v.at[b],
            out_hbm.at[pl.ds(base + c * CL, CL), pl.ds(i * D, D)],
            ssems[b])
        if k + NBUF < n:
            sh[k].wait()
            start_gather(k + NBUF)
    for k in range(max(0, n - NBUF), n):
        sh[k].wait()


def kernel(x, W_count, W_fans, W_stars, W_hot, W_more, W_profile, W_cute,
           W_list, W_writer, W_photos):
    xr = (x.astype(jnp.int32).T
          .reshape(NT, NW, CH, CL)
          .transpose(1, 0, 2, 3))
    return _emb_lookup(xr, W_count, W_fans, W_stars, W_hot, W_more,
                       W_profile, W_cute, W_list, W_writer, W_photos)
